# Initial kernel scaffold; baseline (speedup 1.0000x reference)
import jax
import jax.numpy as jnp
from jax.experimental import pallas as pl


def kernel(x, layer1, layer2, layer3, layer4):
    raise NotImplementedError("write your pallas kernel here")


# speedup vs baseline: 97.4093x; 97.4093x over previous
import functools

import jax
import jax.numpy as jnp
from jax import lax
from jax.experimental import pallas as pl
from jax.experimental.pallas import tpu as pltpu
from jax.experimental.pallas import tpu_sc as plsc

NC, NS, L = 2, 16, 16
NPAIR = NS
NPIX = 8 * 512 * 512
PIX_PER_PAIR = NPIX // NPAIR
CHUNK = 2048
NCHUNK = PIX_PER_PAIR // CHUNK
VREGS = CHUNK // L

L1_HALF_W = 65792
L1_ODD_OFF = 65536
MASK_HI = -65536


def _pack(layer):
    a = layer.reshape(-1).astype(jnp.bfloat16).reshape(-1, 2)
    return lax.bitcast_convert_type(a, jnp.int32)


def _bilerp(tbl, gx, gy, s_f, s_i, row_half, tbl_max, word_off):
    half = s_f * 0.5
    ix = jnp.clip(gx * half + (half - 0.5), 0.0, s_f - 1.0)
    iy = jnp.clip(gy * half + (half - 0.5), 0.0, s_f - 1.0)
    x0i = ix.astype(jnp.int32)
    y0i = iy.astype(jnp.int32)
    wx1 = ix - x0i.astype(jnp.float32)
    wy1 = iy - y0i.astype(jnp.float32)
    wx0 = 1.0 - wx1
    wy0 = 1.0 - wy1
    idx = y0i * s_i + x0i
    p = (idx & 1) == 1
    xmax = x0i == s_i - 1
    ymax = y0i == s_i - 1
    w0 = lax.shift_right_arithmetic(idx, 1) - word_off
    zero = jnp.int32(0)
    w0c = jnp.minimum(jnp.maximum(w0, zero), tbl_max)
    w0b = jnp.minimum(jnp.maximum(w0 + 1, zero), tbl_max)
    w1 = jnp.where(ymax, w0, w0 + row_half)
    w1c = jnp.minimum(jnp.maximum(w1, zero), tbl_max)
    w1b = jnp.minimum(jnp.maximum(w1 + 1, zero), tbl_max)
    g00 = plsc.load_gather(tbl, [w0c])
    g01 = plsc.load_gather(tbl, [w0b])
    g10 = plsc.load_gather(tbl, [w1c])
    g11 = plsc.load_gather(tbl, [w1b])
    hi00 = g00 & MASK_HI
    hi10 = g10 & MASK_HI
    pq = jnp.logical_and(p, jnp.logical_not(xmax))
    bc = lambda v: lax.bitcast_convert_type(v, jnp.float32)
    v00 = bc(jnp.where(p, hi00, lax.shift_left(g00, 16)))
    v01 = bc(jnp.where(pq, lax.shift_left(g01, 16), hi00))
    v10 = bc(jnp.where(p, hi10, lax.shift_left(g10, 16)))
    v11 = bc(jnp.where(pq, lax.shift_left(g11, 16), hi10))
    r = wy0 * (wx0 * v00 + wx1 * v01) + wy1 * (wx0 * v10 + wx1 * v11)
    return r, y0i


def _sc_body(xf, t1p, t2p, t3p, t4p, out_a, out_b, t1s, t23s, t4s, xbuf, obuf):
    member = lax.axis_index("c")
    pair = lax.axis_index("s")

    pltpu.sync_copy(t1p.at[pl.ds(member * L1_ODD_OFF, L1_HALF_W)], t1s)
    @pl.when(member == 0)
    def _():
        pltpu.sync_copy(t2p, t23s.at[pl.ds(0, 32768)])
    @pl.when(member == 1)
    def _():
        pltpu.sync_copy(t3p, t23s.at[pl.ds(0, 8192)])
    pltpu.sync_copy(t4p, t4s)

    f0 = jnp.float32(0.0)
    l1_off = member * L1_ODD_OFF
    l1_max = 65791 - member * 256
    s23f = jnp.where(member == 0, jnp.float32(256.0), jnp.float32(128.0))
    s23i = jnp.where(member == 0, 256, 128)
    row23 = jnp.where(member == 0, 128, 64)
    max23 = jnp.where(member == 0, 32767, 8191)
    m_is_0 = member == 0

    iota = lax.iota(jnp.int32, L)
    idx_e = iota * 2
    idx_o = idx_e + 1

    def chunk_body(ci, carry):
        base = pair * PIX_PER_PAIR + ci * CHUNK
        pltpu.sync_copy(xf.at[pl.ds(base * 2, 2 * CHUNK)], xbuf)

        def vec_body(vi, c2):
            o = vi * (2 * L)
            gx = plsc.load_gather(xbuf, [idx_e + o])
            gy = plsc.load_gather(xbuf, [idx_o + o])
            r1, y0i = _bilerp(t1s, gx, gy, jnp.float32(512.0), 512, 256,
                              l1_max, l1_off)
            hi_side = y0i >= 256
            mine = jnp.logical_xor(hi_side, m_is_0)
            acc = jnp.where(mine, r1, f0)
            r23, _ = _bilerp(t23s, gx, gy, s23f, s23i, row23, max23, 0)
            obuf[pl.ds(vi * L, L)] = acc + r23
            return c2

        lax.fori_loop(0, VREGS, vec_body, 0)

        @pl.when((ci & 1) == member)
        def _():
            def l4_body(vi, c2):
                o = vi * (2 * L)
                gx = plsc.load_gather(xbuf, [idx_e + o])
                gy = plsc.load_gather(xbuf, [idx_o + o])
                r4, _ = _bilerp(t4s, gx, gy, jnp.float32(64.0), 64, 32, 2047, 0)
                sl = pl.ds(vi * L, L)
                obuf[sl] = obuf[sl] + r4
                return c2
            lax.fori_loop(0, VREGS, l4_body, 0)

        @pl.when(member == 0)
        def _():
            pltpu.sync_copy(obuf, out_a.at[pl.ds(base, CHUNK)])
        @pl.when(member == 1)
        def _():
            pltpu.sync_copy(obuf, out_b.at[pl.ds(base, CHUNK)])
        return carry

    lax.fori_loop(0, NCHUNK, chunk_body, 0)


def _add_body(a_ref, b_ref, o_ref):
    o_ref[...] = a_ref[...] + b_ref[...]


def kernel(x, layer1, layer2, layer3, layer4):
    xf = x.reshape(-1)
    t1p = jnp.concatenate([_pack(layer1), jnp.zeros((256,), jnp.int32)])
    t2p = _pack(layer2)
    t3p = _pack(layer3)
    t4p = _pack(layer4)

    mesh = plsc.VectorSubcoreMesh(core_axis_name="c", subcore_axis_name="s")
    sc = pl.kernel(
        _sc_body,
        out_type=(
            jax.ShapeDtypeStruct((NPIX,), jnp.float32),
            jax.ShapeDtypeStruct((NPIX,), jnp.float32),
        ),
        mesh=mesh,
        compiler_params=pltpu.CompilerParams(needs_layout_passes=False),
        scratch_types=[
            pltpu.VMEM((L1_HALF_W,), jnp.int32),
            pltpu.VMEM((32768,), jnp.int32),
            pltpu.VMEM((2048,), jnp.int32),
            pltpu.VMEM((2 * CHUNK,), jnp.float32),
            pltpu.VMEM((CHUNK,), jnp.float32),
        ],
    )
    out_a, out_b = sc(xf, t1p, t2p, t3p, t4p)

    a2 = out_a.reshape(2048, 1024)
    b2 = out_b.reshape(2048, 1024)
    out = pl.pallas_call(
        _add_body,
        grid=(16,),
        in_specs=[
            pl.BlockSpec((128, 1024), lambda i: (i, 0)),
            pl.BlockSpec((128, 1024), lambda i: (i, 0)),
        ],
        out_specs=pl.BlockSpec((128, 1024), lambda i: (i, 0)),
        out_shape=jax.ShapeDtypeStruct((2048, 1024), jnp.float32),
    )(a2, b2)
    return out.reshape(8, 1, 512, 512)


# speedup vs baseline: 509.1666x; 5.2271x over previous
import functools

import jax
import jax.numpy as jnp
from jax import lax
from jax.experimental import pallas as pl
from jax.experimental.pallas import tpu as pltpu
from jax.experimental.pallas import tpu_sc as plsc

NC, NS, L = 2, 16, 16
NPAIR = NS
NPIX = 8 * 512 * 512
CHUNK = 4096
NCHUNK_PER_PAIR = NPIX // CHUNK // NPAIR
VREGS = CHUNK // L

L1_HALF_W = 65792
L1_ODD_OFF = 65536
MASK_HI = -65536


def _pack(layer):
    a = layer.reshape(-1).astype(jnp.bfloat16).reshape(-1, 2)
    return lax.bitcast_convert_type(a, jnp.int32)


def _bilerp(tbl, gx, gy, s_f, s_i, row_half, tbl_max, word_off):
    half = s_f * 0.5
    ix = jnp.clip(gx * half + (half - 0.5), 0.0, s_f - 1.0)
    iy = jnp.clip(gy * half + (half - 0.5), 0.0, s_f - 1.0)
    x0i = ix.astype(jnp.int32)
    y0i = iy.astype(jnp.int32)
    wx1 = ix - x0i.astype(jnp.float32)
    wy1 = iy - y0i.astype(jnp.float32)
    wx0 = 1.0 - wx1
    wy0 = 1.0 - wy1
    idx = y0i * s_i + x0i
    p = (idx & 1) == 1
    xmax = x0i == s_i - 1
    ymax = y0i == s_i - 1
    w0 = lax.shift_right_arithmetic(idx, 1) - word_off
    zero = jnp.int32(0)
    w0c = jnp.minimum(jnp.maximum(w0, zero), tbl_max)
    w0b = jnp.minimum(jnp.maximum(w0 + 1, zero), tbl_max)
    w1 = jnp.where(ymax, w0, w0 + row_half)
    w1c = jnp.minimum(jnp.maximum(w1, zero), tbl_max)
    w1b = jnp.minimum(jnp.maximum(w1 + 1, zero), tbl_max)
    g00 = plsc.load_gather(tbl, [w0c])
    g01 = plsc.load_gather(tbl, [w0b])
    g10 = plsc.load_gather(tbl, [w1c])
    g11 = plsc.load_gather(tbl, [w1b])
    hi00 = g00 & MASK_HI
    hi10 = g10 & MASK_HI
    pq = jnp.logical_and(p, jnp.logical_not(xmax))
    bc = lambda v: lax.bitcast_convert_type(v, jnp.float32)
    v00 = bc(jnp.where(p, hi00, lax.shift_left(g00, 16)))
    v01 = bc(jnp.where(pq, lax.shift_left(g01, 16), hi00))
    v10 = bc(jnp.where(p, hi10, lax.shift_left(g10, 16)))
    v11 = bc(jnp.where(pq, lax.shift_left(g11, 16), hi10))
    r = wy0 * (wx0 * v00 + wx1 * v01) + wy1 * (wx0 * v10 + wx1 * v11)
    return r, y0i


def _vreg_offsets(vi):
    r = lax.shift_right_logical(vi, 5)
    q = vi & 31
    xt = lax.shift_right_logical(q, 3)
    j = q & 7
    goff = r * 1024 + xt * 256 + j * 16
    ooff = xt * 1024 + r * 128 + j * 16
    return goff, ooff


def _sc_body(xf, t1p, t2p, t3p, t4p, out_a, out_b, t1s, t23s, t4s, xbuf, obuf):
    member = lax.axis_index("c")
    pair = lax.axis_index("s")

    pltpu.sync_copy(t1p.at[pl.ds(member * L1_ODD_OFF, L1_HALF_W)], t1s)
    @pl.when(member == 0)
    def _():
        pltpu.sync_copy(t2p, t23s.at[pl.ds(0, 32768)])
    @pl.when(member == 1)
    def _():
        pltpu.sync_copy(t3p, t23s.at[pl.ds(0, 8192)])
    pltpu.sync_copy(t4p, t4s)

    f0 = jnp.float32(0.0)
    l1_off = member * L1_ODD_OFF
    l1_max = 65791 - member * 256
    s23f = jnp.where(member == 0, jnp.float32(256.0), jnp.float32(128.0))
    s23i = jnp.where(member == 0, 256, 128)
    row23 = jnp.where(member == 0, 128, 64)
    max23 = jnp.where(member == 0, 32767, 8191)
    m_is_0 = member == 0

    def chunk_body(ci, carry):
        chunk_id = pair * NCHUNK_PER_PAIR + ci
        base = chunk_id * CHUNK
        pltpu.sync_copy(xf.at[pl.ds(base * 2, 2 * CHUNK)], xbuf)

        def vec_body(vi, c2):
            goff, ooff = _vreg_offsets(vi)
            gx = xbuf[pl.ds(goff, L)]
            gy = xbuf[pl.ds(goff + 128, L)]
            r1, y0i = _bilerp(t1s, gx, gy, jnp.float32(512.0), 512, 256,
                              l1_max, l1_off)
            hi_side = y0i >= 256
            mine = jnp.logical_xor(hi_side, m_is_0)
            acc = jnp.where(mine, r1, f0)
            r23, _ = _bilerp(t23s, gx, gy, s23f, s23i, row23, max23, 0)
            obuf[pl.ds(ooff, L)] = acc + r23
            return c2

        lax.fori_loop(0, VREGS, vec_body, 0)

        @pl.when((ci & 1) == member)
        def _():
            def l4_body(vi, c2):
                goff, ooff = _vreg_offsets(vi)
                gx = xbuf[pl.ds(goff, L)]
                gy = xbuf[pl.ds(goff + 128, L)]
                r4, _ = _bilerp(t4s, gx, gy, jnp.float32(64.0), 64, 32, 2047, 0)
                sl = pl.ds(ooff, L)
                obuf[sl] = obuf[sl] + r4
                return c2
            lax.fori_loop(0, VREGS, l4_body, 0)

        @pl.when(member == 0)
        def _():
            pltpu.sync_copy(obuf, out_a.at[pl.ds(base, CHUNK)])
        @pl.when(member == 1)
        def _():
            pltpu.sync_copy(obuf, out_b.at[pl.ds(base, CHUNK)])
        return carry

    lax.fori_loop(0, NCHUNK_PER_PAIR, chunk_body, 0)


def _add_body(a_ref, b_ref, o_ref):
    o_ref[...] = a_ref[...] + b_ref[...]


def kernel(x, layer1, layer2, layer3, layer4):
    xf = x.reshape(8, 512, 4, 128, 2).transpose(0, 1, 2, 4, 3).reshape(-1)
    t1p = jnp.concatenate([_pack(layer1), jnp.zeros((256,), jnp.int32)])
    t2p = _pack(layer2)
    t3p = _pack(layer3)
    t4p = _pack(layer4)

    mesh = plsc.VectorSubcoreMesh(core_axis_name="c", subcore_axis_name="s")
    sc = pl.kernel(
        _sc_body,
        out_type=(
            jax.ShapeDtypeStruct((NPIX,), jnp.float32),
            jax.ShapeDtypeStruct((NPIX,), jnp.float32),
        ),
        mesh=mesh,
        compiler_params=pltpu.CompilerParams(needs_layout_passes=False),
        scratch_types=[
            pltpu.VMEM((L1_HALF_W,), jnp.int32),
            pltpu.VMEM((32768,), jnp.int32),
            pltpu.VMEM((2048,), jnp.int32),
            pltpu.VMEM((2 * CHUNK,), jnp.float32),
            pltpu.VMEM((CHUNK,), jnp.float32),
        ],
    )
    out_a, out_b = sc(xf, t1p, t2p, t3p, t4p)

    a2 = out_a.reshape(16384, 128)
    b2 = out_b.reshape(16384, 128)
    s2 = pl.pallas_call(
        _add_body,
        grid=(16,),
        in_specs=[
            pl.BlockSpec((1024, 128), lambda i: (i, 0)),
            pl.BlockSpec((1024, 128), lambda i: (i, 0)),
        ],
        out_specs=pl.BlockSpec((1024, 128), lambda i: (i, 0)),
        out_shape=jax.ShapeDtypeStruct((16384, 128), jnp.float32),
    )(a2, b2)
    return (s2.reshape(8, 64, 4, 8, 128)
              .transpose(0, 1, 3, 2, 4)
              .reshape(8, 1, 512, 512))


# speedup vs baseline: 716.5689x; 1.4073x over previous
import functools

import jax
import jax.numpy as jnp
from jax import lax
from jax.experimental import pallas as pl
from jax.experimental.pallas import tpu as pltpu
from jax.experimental.pallas import tpu_sc as plsc

NC, NS, L = 2, 16, 16
NPAIR = NS
NPIX = 8 * 512 * 512
CHUNK = 4096
NCHUNK_PER_PAIR = NPIX // CHUNK // NPAIR
VREGS = CHUNK // L

L1_HALF_W = 65792
L1_ODD_OFF = 65536
MASK_HI = -65536


def _pack(layer):
    a = layer.reshape(-1).astype(jnp.bfloat16).reshape(-1, 2)
    return lax.bitcast_convert_type(a, jnp.int32)


def _bilerp(tbl, gx, gy, s_f, s_i, row_half, tbl_max, word_off, split=False):
    half = s_f * 0.5
    ix = jnp.clip(gx * half + (half - 0.5), 0.0, s_f - 1.0)
    iy = jnp.clip(gy * half + (half - 0.5), 0.0, s_f - 1.0)
    x0i = ix.astype(jnp.int32)
    y0i = iy.astype(jnp.int32)
    wx1 = ix - x0i.astype(jnp.float32)
    wy1 = iy - y0i.astype(jnp.float32)
    idx = y0i * s_i + x0i
    p = (idx & 1) == 1
    xmax = x0i == s_i - 1
    ymax = y0i == s_i - 1
    w0 = lax.shift_right_arithmetic(idx, 1)
    if split:
        zero = jnp.int32(0)
        w0 = w0 - word_off
        w0c = jnp.minimum(jnp.maximum(w0, zero), tbl_max)
        w0b = jnp.minimum(jnp.maximum(w0 + 1, zero), tbl_max)
        w1 = jnp.where(ymax, w0, w0 + row_half)
        w1c = jnp.minimum(jnp.maximum(w1, zero), tbl_max)
        w1b = jnp.minimum(jnp.maximum(w1 + 1, zero), tbl_max)
    else:
        w0c = w0
        w0b = jnp.minimum(w0 + 1, tbl_max)
        w1 = jnp.where(ymax, w0, w0 + row_half)
        w1c = w1
        w1b = jnp.minimum(w1 + 1, tbl_max)
    g00 = plsc.load_gather(tbl, [w0c])
    g01 = plsc.load_gather(tbl, [w0b])
    g10 = plsc.load_gather(tbl, [w1c])
    g11 = plsc.load_gather(tbl, [w1b])
    hi00 = g00 & MASK_HI
    hi10 = g10 & MASK_HI
    pq = jnp.logical_and(p, jnp.logical_not(xmax))
    bc = lambda v: lax.bitcast_convert_type(v, jnp.float32)
    v00 = bc(jnp.where(p, hi00, lax.shift_left(g00, 16)))
    v01 = bc(jnp.where(pq, lax.shift_left(g01, 16), hi00))
    v10 = bc(jnp.where(p, hi10, lax.shift_left(g10, 16)))
    v11 = bc(jnp.where(pq, lax.shift_left(g11, 16), hi10))
    top = v00 + wx1 * (v01 - v00)
    bot = v10 + wx1 * (v11 - v10)
    r = top + wy1 * (bot - top)
    return r, y0i


def _vreg_offsets(vi):
    r = lax.shift_right_logical(vi, 5)
    q = vi & 31
    xt = lax.shift_right_logical(q, 3)
    j = q & 7
    goff = r * 1024 + xt * 256 + j * 16
    ooff = xt * 1024 + r * 128 + j * 16
    return goff, ooff


def _sc_body(xf, t1p, t2p, t3p, t4p, out_a, out_b,
             t1s, t23s, t4s, xbuf, obuf, sin, sout):
    member = lax.axis_index("c")
    pair = lax.axis_index("s")

    pltpu.sync_copy(t1p.at[pl.ds(member * L1_ODD_OFF, L1_HALF_W)], t1s)
    @pl.when(member == 0)
    def _():
        pltpu.sync_copy(t2p, t23s.at[pl.ds(0, 32768)])
    @pl.when(member == 1)
    def _():
        pltpu.sync_copy(t3p, t23s.at[pl.ds(0, 8192)])
    pltpu.sync_copy(t4p, t4s)

    f0 = jnp.float32(0.0)
    l1_off = member * L1_ODD_OFF
    l1_max = 65791 - member * 256
    s23f = jnp.where(member == 0, jnp.float32(256.0), jnp.float32(128.0))
    s23i = jnp.where(member == 0, 256, 128)
    row23 = jnp.where(member == 0, 128, 64)
    max23 = jnp.where(member == 0, 32767, 8191)
    m_is_0 = member == 0

    def in_start(ci, b):
        base2 = (pair * NCHUNK_PER_PAIR + ci) * CHUNK * 2
        pltpu.async_copy(xf.at[pl.ds(base2, 2 * CHUNK)],
                         xbuf.at[pl.ds(b * 2 * CHUNK, 2 * CHUNK)], sin.at[b])

    def in_wait(b):
        pltpu.make_async_copy(
            xf.at[pl.ds(0, 2 * CHUNK)],
            xbuf.at[pl.ds(b * 2 * CHUNK, 2 * CHUNK)], sin.at[b]).wait()

    def out_start(ci, b):
        base = (pair * NCHUNK_PER_PAIR + ci) * CHUNK
        ob = obuf.at[pl.ds(b * CHUNK, CHUNK)]
        @pl.when(member == 0)
        def _():
            pltpu.async_copy(ob, out_a.at[pl.ds(base, CHUNK)], sout.at[b])
        @pl.when(member == 1)
        def _():
            pltpu.async_copy(ob, out_b.at[pl.ds(base, CHUNK)], sout.at[b])

    def out_wait(b):
        pltpu.make_async_copy(
            obuf.at[pl.ds(b * CHUNK, CHUNK)],
            out_a.at[pl.ds(0, CHUNK)], sout.at[b]).wait()

    def compute(ci, b):
        xo = b * 2 * CHUNK
        oo = b * CHUNK

        @plsc.parallel_loop(0, VREGS, unroll=2)
        def _(vi):
            goff, ooff = _vreg_offsets(vi)
            gx = xbuf[pl.ds(xo + goff, L)]
            gy = xbuf[pl.ds(xo + goff + 128, L)]
            r1, y0i = _bilerp(t1s, gx, gy, jnp.float32(512.0), 512, 256,
                              l1_max, l1_off, split=True)
            hi_side = y0i >= 256
            mine = jnp.logical_xor(hi_side, m_is_0)
            acc = jnp.where(mine, r1, f0)
            r23, _ = _bilerp(t23s, gx, gy, s23f, s23i, row23, max23, 0)
            obuf[pl.ds(oo + ooff, L)] = acc + r23

        @pl.when((ci & 1) == member)
        def _():
            @plsc.parallel_loop(0, VREGS, unroll=2)
            def _(vi):
                goff, ooff = _vreg_offsets(vi)
                gx = xbuf[pl.ds(xo + goff, L)]
                gy = xbuf[pl.ds(xo + goff + 128, L)]
                r4, _ = _bilerp(t4s, gx, gy, jnp.float32(64.0), 64, 32, 2047, 0)
                sl = pl.ds(oo + ooff, L)
                obuf[sl] = obuf[sl] + r4

    in_start(0, 0)

    def outer(cg, carry):
        for b in range(2):
            ci = cg * 2 + b
            in_wait(b)
            @pl.when(ci + 1 < NCHUNK_PER_PAIR)
            def _():
                in_start(ci + 1, 1 - b)
            @pl.when(ci >= 2)
            def _():
                out_wait(b)
            compute(ci, b)
            out_start(ci, b)
        return carry

    lax.fori_loop(0, NCHUNK_PER_PAIR // 2, outer, 0)
    out_wait(0)
    out_wait(1)


def _add_body(a_ref, b_ref, o_ref):
    o_ref[...] = a_ref[...] + b_ref[...]


def kernel(x, layer1, layer2, layer3, layer4):
    xf = x.reshape(8, 512, 4, 128, 2).transpose(0, 1, 2, 4, 3).reshape(-1)
    t1p = jnp.concatenate([_pack(layer1), jnp.zeros((256,), jnp.int32)])
    t2p = _pack(layer2)
    t3p = _pack(layer3)
    t4p = _pack(layer4)

    mesh = plsc.VectorSubcoreMesh(core_axis_name="c", subcore_axis_name="s")
    sc = pl.kernel(
        _sc_body,
        out_type=(
            jax.ShapeDtypeStruct((NPIX,), jnp.float32),
            jax.ShapeDtypeStruct((NPIX,), jnp.float32),
        ),
        mesh=mesh,
        compiler_params=pltpu.CompilerParams(needs_layout_passes=False),
        scratch_types=[
            pltpu.VMEM((L1_HALF_W,), jnp.int32),
            pltpu.VMEM((32768,), jnp.int32),
            pltpu.VMEM((2048,), jnp.int32),
            pltpu.VMEM((4 * CHUNK,), jnp.float32),
            pltpu.VMEM((2 * CHUNK,), jnp.float32),
            pltpu.SemaphoreType.DMA((2,)),
            pltpu.SemaphoreType.DMA((2,)),
        ],
    )
    out_a, out_b = sc(xf, t1p, t2p, t3p, t4p)

    a2 = out_a.reshape(16384, 128)
    b2 = out_b.reshape(16384, 128)
    s2 = pl.pallas_call(
        _add_body,
        grid=(16,),
        in_specs=[
            pl.BlockSpec((1024, 128), lambda i: (i, 0)),
            pl.BlockSpec((1024, 128), lambda i: (i, 0)),
        ],
        out_specs=pl.BlockSpec((1024, 128), lambda i: (i, 0)),
        out_shape=jax.ShapeDtypeStruct((16384, 128), jnp.float32),
    )(a2, b2)
    return (s2.reshape(8, 64, 4, 8, 128)
              .transpose(0, 1, 3, 2, 4)
              .reshape(8, 1, 512, 512))


# speedup vs baseline: 736.4342x; 1.0277x over previous
import functools

import jax
import jax.numpy as jnp
from jax import lax
from jax.experimental import pallas as pl
from jax.experimental.pallas import tpu as pltpu
from jax.experimental.pallas import tpu_sc as plsc

NC, NS, L = 2, 16, 16
NPAIR = NS
NPIX = 8 * 512 * 512
CHUNK = 4096
NCHUNK_PER_PAIR = NPIX // CHUNK // NPAIR
VREGS = CHUNK // L

L1_HALF_W = 65792
L1_ODD_OFF = 65536
MASK_HI = -65536


def _pack(layer):
    a = layer.reshape(-1).astype(jnp.bfloat16).reshape(-1, 2)
    return lax.bitcast_convert_type(a, jnp.int32)


def _bilerp(tbl, gx, gy, s_f, s_i, row_half, tbl_max, word_off, split=False):
    half = s_f * 0.5
    ix = jnp.clip(gx * half + (half - 0.5), 0.0, s_f - 1.0)
    iy = jnp.clip(gy * half + (half - 0.5), 0.0, s_f - 1.0)
    x0i = ix.astype(jnp.int32)
    y0i = iy.astype(jnp.int32)
    wx1 = ix - x0i.astype(jnp.float32)
    wy1 = iy - y0i.astype(jnp.float32)
    idx = y0i * s_i + x0i
    p = (idx & 1) == 1
    xmax = x0i == s_i - 1
    ymax = y0i == s_i - 1
    w0 = lax.shift_right_arithmetic(idx, 1)
    if split:
        zero = jnp.int32(0)
        w0 = w0 - word_off
        w0c = jnp.minimum(jnp.maximum(w0, zero), tbl_max)
        w0b = jnp.minimum(jnp.maximum(w0 + 1, zero), tbl_max)
        w1 = jnp.where(ymax, w0, w0 + row_half)
        w1c = jnp.minimum(jnp.maximum(w1, zero), tbl_max)
        w1b = jnp.minimum(jnp.maximum(w1 + 1, zero), tbl_max)
    else:
        w0c = w0
        w0b = jnp.minimum(w0 + 1, tbl_max)
        w1 = jnp.where(ymax, w0, w0 + row_half)
        w1c = w1
        w1b = jnp.minimum(w1 + 1, tbl_max)
    g00 = plsc.load_gather(tbl, [w0c])
    g01 = plsc.load_gather(tbl, [w0b])
    g10 = plsc.load_gather(tbl, [w1c])
    g11 = plsc.load_gather(tbl, [w1b])
    hi00 = g00 & MASK_HI
    hi10 = g10 & MASK_HI
    pq = jnp.logical_and(p, jnp.logical_not(xmax))
    bc = lambda v: lax.bitcast_convert_type(v, jnp.float32)
    v00 = bc(jnp.where(p, hi00, lax.shift_left(g00, 16)))
    v01 = bc(jnp.where(pq, lax.shift_left(g01, 16), hi00))
    v10 = bc(jnp.where(p, hi10, lax.shift_left(g10, 16)))
    v11 = bc(jnp.where(pq, lax.shift_left(g11, 16), hi10))
    top = v00 + wx1 * (v01 - v00)
    bot = v10 + wx1 * (v11 - v10)
    r = top + wy1 * (bot - top)
    return r, y0i


def _vreg_offsets(vi):
    r = lax.shift_right_logical(vi, 5)
    q = vi & 31
    xt = lax.shift_right_logical(q, 3)
    j = q & 7
    goff = r * 1024 + xt * 256 + j * 16
    ooff = xt * 1024 + r * 128 + j * 16
    return goff, ooff


def _sc_body(xf, t1p, t2p, t3p, t4p, out_a, out_b,
             t1s, t23s, t4s, xbuf, obuf, sin, sout):
    member = lax.axis_index("c")
    pair = lax.axis_index("s")

    pltpu.sync_copy(t1p.at[pl.ds(member * L1_ODD_OFF, L1_HALF_W)], t1s)
    @pl.when(member == 0)
    def _():
        pltpu.sync_copy(t2p, t23s.at[pl.ds(0, 32768)])
    @pl.when(member == 1)
    def _():
        pltpu.sync_copy(t3p, t23s.at[pl.ds(0, 8192)])
    pltpu.sync_copy(t4p, t4s)

    f0 = jnp.float32(0.0)
    l1_off = member * L1_ODD_OFF
    l1_max = 65791 - member * 256
    s23f = jnp.where(member == 0, jnp.float32(256.0), jnp.float32(128.0))
    s23i = jnp.where(member == 0, 256, 128)
    row23 = jnp.where(member == 0, 128, 64)
    max23 = jnp.where(member == 0, 32767, 8191)
    m_is_0 = member == 0

    def in_start(ci, b):
        base2 = (pair * NCHUNK_PER_PAIR + ci) * CHUNK * 2
        pltpu.async_copy(xf.at[pl.ds(base2, 2 * CHUNK)],
                         xbuf.at[pl.ds(b * 2 * CHUNK, 2 * CHUNK)], sin.at[b])

    def in_wait(b):
        pltpu.make_async_copy(
            xf.at[pl.ds(0, 2 * CHUNK)],
            xbuf.at[pl.ds(b * 2 * CHUNK, 2 * CHUNK)], sin.at[b]).wait()

    def out_start(ci, b):
        base = (pair * NCHUNK_PER_PAIR + ci) * CHUNK
        ob = obuf.at[pl.ds(b * CHUNK, CHUNK)]
        @pl.when(member == 0)
        def _():
            pltpu.async_copy(ob, out_a.at[pl.ds(base, CHUNK)], sout.at[b])
        @pl.when(member == 1)
        def _():
            pltpu.async_copy(ob, out_b.at[pl.ds(base, CHUNK)], sout.at[b])

    def out_wait(b):
        pltpu.make_async_copy(
            obuf.at[pl.ds(b * CHUNK, CHUNK)],
            out_a.at[pl.ds(0, CHUNK)], sout.at[b]).wait()

    def compute(ci, b):
        xo = b * 2 * CHUNK
        oo = b * CHUNK

        @plsc.parallel_loop(0, VREGS, unroll=4)
        def _(vi):
            goff, ooff = _vreg_offsets(vi)
            gx = xbuf[pl.ds(xo + goff, L)]
            gy = xbuf[pl.ds(xo + goff + 128, L)]
            r1, y0i = _bilerp(t1s, gx, gy, jnp.float32(512.0), 512, 256,
                              l1_max, l1_off, split=True)
            hi_side = y0i >= 256
            mine = jnp.logical_xor(hi_side, m_is_0)
            acc = jnp.where(mine, r1, f0)
            r23, _ = _bilerp(t23s, gx, gy, s23f, s23i, row23, max23, 0)
            obuf[pl.ds(oo + ooff, L)] = acc + r23

        @pl.when((ci & 1) == member)
        def _():
            @plsc.parallel_loop(0, VREGS, unroll=4)
            def _(vi):
                goff, ooff = _vreg_offsets(vi)
                gx = xbuf[pl.ds(xo + goff, L)]
                gy = xbuf[pl.ds(xo + goff + 128, L)]
                r4, _ = _bilerp(t4s, gx, gy, jnp.float32(64.0), 64, 32, 2047, 0)
                sl = pl.ds(oo + ooff, L)
                obuf[sl] = obuf[sl] + r4

    in_start(0, 0)

    def outer(cg, carry):
        for b in range(2):
            ci = cg * 2 + b
            in_wait(b)
            @pl.when(ci + 1 < NCHUNK_PER_PAIR)
            def _():
                in_start(ci + 1, 1 - b)
            @pl.when(ci >= 2)
            def _():
                out_wait(b)
            compute(ci, b)
            out_start(ci, b)
        return carry

    lax.fori_loop(0, NCHUNK_PER_PAIR // 2, outer, 0)
    out_wait(0)
    out_wait(1)


def _add_body(a_ref, b_ref, o_ref):
    o_ref[...] = a_ref[...] + b_ref[...]


def kernel(x, layer1, layer2, layer3, layer4):
    xf = x.reshape(8, 512, 4, 128, 2).transpose(0, 1, 2, 4, 3).reshape(-1)
    t1p = jnp.concatenate([_pack(layer1), jnp.zeros((256,), jnp.int32)])
    t2p = _pack(layer2)
    t3p = _pack(layer3)
    t4p = _pack(layer4)

    mesh = plsc.VectorSubcoreMesh(core_axis_name="c", subcore_axis_name="s")
    sc = pl.kernel(
        _sc_body,
        out_type=(
            jax.ShapeDtypeStruct((NPIX,), jnp.float32),
            jax.ShapeDtypeStruct((NPIX,), jnp.float32),
        ),
        mesh=mesh,
        compiler_params=pltpu.CompilerParams(needs_layout_passes=False),
        scratch_types=[
            pltpu.VMEM((L1_HALF_W,), jnp.int32),
            pltpu.VMEM((32768,), jnp.int32),
            pltpu.VMEM((2048,), jnp.int32),
            pltpu.VMEM((4 * CHUNK,), jnp.float32),
            pltpu.VMEM((2 * CHUNK,), jnp.float32),
            pltpu.SemaphoreType.DMA((2,)),
            pltpu.SemaphoreType.DMA((2,)),
        ],
    )
    out_a, out_b = sc(xf, t1p, t2p, t3p, t4p)

    a2 = out_a.reshape(16384, 128)
    b2 = out_b.reshape(16384, 128)
    s2 = pl.pallas_call(
        _add_body,
        grid=(16,),
        in_specs=[
            pl.BlockSpec((1024, 128), lambda i: (i, 0)),
            pl.BlockSpec((1024, 128), lambda i: (i, 0)),
        ],
        out_specs=pl.BlockSpec((1024, 128), lambda i: (i, 0)),
        out_shape=jax.ShapeDtypeStruct((16384, 128), jnp.float32),
    )(a2, b2)
    return (s2.reshape(8, 64, 4, 8, 128)
              .transpose(0, 1, 3, 2, 4)
              .reshape(8, 1, 512, 512))


# speedup vs baseline: 930.4959x; 1.2635x over previous
import functools

import jax
import jax.numpy as jnp
from jax import lax
from jax.experimental import pallas as pl
from jax.experimental.pallas import tpu as pltpu
from jax.experimental.pallas import tpu_sc as plsc

NC, NS, L = 2, 16, 16
NPAIR = NS
NPIX = 8 * 512 * 512
CHUNK = 4096
NCHUNK_PER_PAIR = NPIX // CHUNK // NPAIR
VREGS = CHUNK // L

L1_HALF_W = 65792
L1_ODD_OFF = 65536
MASK_HI = -65536


def _pack(layer):
    a = layer.reshape(-1).astype(jnp.bfloat16).reshape(-1, 2)
    return lax.bitcast_convert_type(a, jnp.int32)


def _bilerp(tbl, gx, gy, s_f, s_i, row_half, tbl_max, word_off, split=False):
    half = s_f * 0.5
    ix = jnp.clip(gx * half + (half - 0.5), 0.0, s_f - 1.0)
    iy = jnp.clip(gy * half + (half - 0.5), 0.0, s_f - 1.0)
    x0i = ix.astype(jnp.int32)
    y0i = iy.astype(jnp.int32)
    wx1 = ix - x0i.astype(jnp.float32)
    wy1 = iy - y0i.astype(jnp.float32)
    idx = y0i * s_i + x0i
    p = (idx & 1) == 1
    xmax = x0i == s_i - 1
    ymax = y0i == s_i - 1
    w0 = lax.shift_right_arithmetic(idx, 1)
    if split:
        zero = jnp.int32(0)
        w0 = w0 - word_off
        w0c = jnp.minimum(jnp.maximum(w0, zero), tbl_max)
        w0b = jnp.minimum(jnp.maximum(w0 + 1, zero), tbl_max)
        w1 = jnp.where(ymax, w0, w0 + row_half)
        w1c = jnp.minimum(jnp.maximum(w1, zero), tbl_max)
        w1b = jnp.minimum(jnp.maximum(w1 + 1, zero), tbl_max)
    else:
        w0c = w0
        w0b = jnp.minimum(w0 + 1, tbl_max)
        w1 = jnp.where(ymax, w0, w0 + row_half)
        w1c = w1
        w1b = jnp.minimum(w1 + 1, tbl_max)
    g00 = plsc.load_gather(tbl, [w0c])
    g01 = plsc.load_gather(tbl, [w0b])
    g10 = plsc.load_gather(tbl, [w1c])
    g11 = plsc.load_gather(tbl, [w1b])
    hi00 = g00 & MASK_HI
    hi10 = g10 & MASK_HI
    pq = jnp.logical_and(p, jnp.logical_not(xmax))
    bc = lambda v: lax.bitcast_convert_type(v, jnp.float32)
    v00 = bc(jnp.where(p, hi00, lax.shift_left(g00, 16)))
    v01 = bc(jnp.where(pq, lax.shift_left(g01, 16), hi00))
    v10 = bc(jnp.where(p, hi10, lax.shift_left(g10, 16)))
    v11 = bc(jnp.where(pq, lax.shift_left(g11, 16), hi10))
    top = v00 + wx1 * (v01 - v00)
    bot = v10 + wx1 * (v11 - v10)
    r = top + wy1 * (bot - top)
    return r, y0i


def _vreg_offsets(vi):
    r = lax.shift_right_logical(vi, 5)
    q = vi & 31
    xt = lax.shift_right_logical(q, 3)
    j = q & 7
    goff = r * 1024 + xt * 256 + j * 16
    ooff = xt * 1024 + r * 128 + j * 16
    return goff, ooff


def _sc_body(xf, l1f, t2p, t3p, t4p, out_a, out_b,
             t1s, t23s, t4s, xbuf, obuf, spl1, sin, sout):
    member = lax.axis_index("c")
    pair = lax.axis_index("s")

    iota = lax.iota(jnp.int32, L)

    def pack_group(kk, dst_off, row0):
        k = kk * L + iota
        xw = k & 255
        r = lax.shift_right_logical(k, 8) + row0
        pos = (lax.shift_right_logical(xw, 6) * 1024 + r * 128
               + (xw & 63) * 2)
        ev = plsc.load_gather(xbuf, [pos])
        od = plsc.load_gather(xbuf, [pos + 1])
        t23s[pl.ds(dst_off + kk * L, L)] = plsc.bitcast(
            plsc.pack(ev, od, format=plsc.PackFormat.INTERLEAVED), jnp.int32)

    for gg in range(2):
        g = member * 32 + pair * 2 + gg
        pltpu.sync_copy(l1f.at[pl.ds(g * 4096, 4096)],
                        xbuf.at[pl.ds(0, 4096)])

        @plsc.parallel_loop(0, 128, unroll=2)
        def _(kk):
            pack_group(kk, gg * 2048, 0)

    pltpu.sync_copy(t23s.at[pl.ds(0, 4096)], spl1.at[pl.ds(pair * 4096, 4096)])

    @pl.when(jnp.logical_and(member == 0, pair == 0))
    def _():
        pltpu.sync_copy(l1f.at[pl.ds(32 * 4096, 4096)],
                        xbuf.at[pl.ds(0, 4096)])

        @plsc.parallel_loop(0, 16)
        def _(kk):
            pack_group(kk, 0, 0)
        pltpu.sync_copy(t23s.at[pl.ds(0, 256)], spl1.at[pl.ds(65536, 256)])

    plsc.subcore_barrier()
    pltpu.sync_copy(spl1, t1s)
    @pl.when(member == 0)
    def _():
        pltpu.sync_copy(t2p, t23s.at[pl.ds(0, 32768)])
    @pl.when(member == 1)
    def _():
        pltpu.sync_copy(t3p, t23s.at[pl.ds(0, 8192)])
    pltpu.sync_copy(t4p, t4s)

    f0 = jnp.float32(0.0)
    l1_off = member * L1_ODD_OFF
    l1_max = 65791 - member * 256
    s23f = jnp.where(member == 0, jnp.float32(256.0), jnp.float32(128.0))
    s23i = jnp.where(member == 0, 256, 128)
    row23 = jnp.where(member == 0, 128, 64)
    max23 = jnp.where(member == 0, 32767, 8191)
    m_is_0 = member == 0

    def in_start(ci, b):
        base2 = (pair * NCHUNK_PER_PAIR + ci) * CHUNK * 2
        pltpu.async_copy(xf.at[pl.ds(base2, 2 * CHUNK)],
                         xbuf.at[pl.ds(b * 2 * CHUNK, 2 * CHUNK)], sin.at[b])

    def in_wait(b):
        pltpu.make_async_copy(
            xf.at[pl.ds(0, 2 * CHUNK)],
            xbuf.at[pl.ds(b * 2 * CHUNK, 2 * CHUNK)], sin.at[b]).wait()

    def out_start(ci, b):
        base = (pair * NCHUNK_PER_PAIR + ci) * CHUNK
        ob = obuf.at[pl.ds(b * CHUNK, CHUNK)]
        @pl.when(member == 0)
        def _():
            pltpu.async_copy(ob, out_a.at[pl.ds(base, CHUNK)], sout.at[b])
        @pl.when(member == 1)
        def _():
            pltpu.async_copy(ob, out_b.at[pl.ds(base, CHUNK)], sout.at[b])

    def out_wait(b):
        pltpu.make_async_copy(
            obuf.at[pl.ds(b * CHUNK, CHUNK)],
            out_a.at[pl.ds(0, CHUNK)], sout.at[b]).wait()

    def compute(ci, b):
        xo = b * 2 * CHUNK
        oo = b * CHUNK

        @plsc.parallel_loop(0, VREGS, unroll=4)
        def _(vi):
            goff, ooff = _vreg_offsets(vi)
            gx = xbuf[pl.ds(xo + goff, L)]
            gy = xbuf[pl.ds(xo + goff + 128, L)]
            r1, y0i = _bilerp(t1s, gx, gy, jnp.float32(512.0), 512, 256,
                              l1_max, l1_off, split=True)
            hi_side = y0i >= 256
            mine = jnp.logical_xor(hi_side, m_is_0)
            acc = jnp.where(mine, r1, f0)
            r23, _ = _bilerp(t23s, gx, gy, s23f, s23i, row23, max23, 0)
            obuf[pl.ds(oo + ooff, L)] = acc + r23

        @pl.when((ci & 1) == member)
        def _():
            @plsc.parallel_loop(0, VREGS, unroll=4)
            def _(vi):
                goff, ooff = _vreg_offsets(vi)
                gx = xbuf[pl.ds(xo + goff, L)]
                gy = xbuf[pl.ds(xo + goff + 128, L)]
                r4, _ = _bilerp(t4s, gx, gy, jnp.float32(64.0), 64, 32, 2047, 0)
                sl = pl.ds(oo + ooff, L)
                obuf[sl] = obuf[sl] + r4

    in_start(0, 0)

    def outer(cg, carry):
        for b in range(2):
            ci = cg * 2 + b
            in_wait(b)
            @pl.when(ci + 1 < NCHUNK_PER_PAIR)
            def _():
                in_start(ci + 1, 1 - b)
            @pl.when(ci >= 2)
            def _():
                out_wait(b)
            compute(ci, b)
            out_start(ci, b)
        return carry

    lax.fori_loop(0, NCHUNK_PER_PAIR // 2, outer, 0)
    out_wait(0)
    out_wait(1)


def _add_body(a_ref, b_ref, o_ref):
    o_ref[...] = a_ref[...] + b_ref[...]


def kernel(x, layer1, layer2, layer3, layer4):
    xf = x.reshape(8, 512, 4, 128, 2).transpose(0, 1, 2, 4, 3).reshape(-1)
    l1f = (layer1.reshape(64, 8, 4, 128).transpose(0, 2, 1, 3).reshape(-1))
    t2p = _pack(layer2)
    t3p = _pack(layer3)
    t4p = _pack(layer4)

    mesh = plsc.VectorSubcoreMesh(core_axis_name="c", subcore_axis_name="s")
    sc = pl.kernel(
        _sc_body,
        out_type=(
            jax.ShapeDtypeStruct((NPIX,), jnp.float32),
            jax.ShapeDtypeStruct((NPIX,), jnp.float32),
        ),
        mesh=mesh,
        compiler_params=pltpu.CompilerParams(needs_layout_passes=False),
        scratch_types=[
            pltpu.VMEM((L1_HALF_W,), jnp.int32),
            pltpu.VMEM((32768,), jnp.int32),
            pltpu.VMEM((2048,), jnp.int32),
            pltpu.VMEM((4 * CHUNK,), jnp.float32),
            pltpu.VMEM((2 * CHUNK,), jnp.float32),
            pltpu.VMEM_SHARED((L1_HALF_W,), jnp.int32),
            pltpu.SemaphoreType.DMA((2,)),
            pltpu.SemaphoreType.DMA((2,)),
        ],
    )
    out_a, out_b = sc(xf, l1f, t2p, t3p, t4p)

    a2 = out_a.reshape(16384, 128)
    b2 = out_b.reshape(16384, 128)
    s2 = pl.pallas_call(
        _add_body,
        grid=(16,),
        in_specs=[
            pl.BlockSpec((1024, 128), lambda i: (i, 0)),
            pl.BlockSpec((1024, 128), lambda i: (i, 0)),
        ],
        out_specs=pl.BlockSpec((1024, 128), lambda i: (i, 0)),
        out_shape=jax.ShapeDtypeStruct((16384, 128), jnp.float32),
    )(a2, b2)
    return (s2.reshape(8, 64, 4, 8, 128)
              .transpose(0, 1, 3, 2, 4)
              .reshape(8, 1, 512, 512))
